# baseline (speedup 1.0000x reference)
import functools

import jax
import jax.numpy as jnp
from jax import lax
from jax.experimental import pallas as pl
from jax.experimental.pallas import tpu as pltpu
from jax.experimental.pallas import tpu_sc as plsc

NUM_CLASSES = 1203
C_PAD = 1280
NC, NS, L = 2, 16, 16
NW = NC * NS
NSTREAM = 2


def _sc_hist_body(cat_hbm, out_hbm, data, m0, m1, hist, sem,
                  *, rows_per_lane, ann):
    markers = (m0, m1)
    wid = lax.axis_index("s") * NC + lax.axis_index("c")
    per_tile = rows_per_lane * ann * L
    copy = pltpu.async_copy(
        cat_hbm.at[pl.ds(wid * per_tile, per_tile)], data, sem)

    iota = lax.iota(jnp.int32, 16)
    ones = jnp.ones((16,), jnp.int32)
    neg1 = jnp.full((16,), -1, jnp.int32)
    zero = jnp.zeros((16,), jnp.int32)

    def init_body(i, carry):
        for u in range(8):
            for m in markers:
                m[pl.ds(i * 128 + u * 16, 16)] = neg1
        return carry

    lax.fori_loop(0, L * C_PAD // 128, init_body, 0)

    def hinit_body(i, carry):
        for u in range(8):
            hist[pl.ds(i * 128 + u * 16, 16)] = zero
        return carry

    lax.fori_loop(0, C_PAD // 128, hinit_body, 0)
    copy.wait()

    quarter = rows_per_lane // NSTREAM
    per_lane = rows_per_lane * ann
    lane_data = iota * per_lane

    def row_body(r, carry):
        rowv = (iota + r) & (quarter - 1)
        stamp = rowv
        dbase = [lane_data + (s * quarter) * ann + rowv * ann
                 for s in range(NSTREAM)]
        for j in range(ann):
            cs = [plsc.load_gather(data, [dbase[s] + j])
                  for s in range(NSTREAM)]
            midx = [(c << 4) + iota for c in cs]
            olds = [plsc.load_gather(markers[s], [midx[s]])
                    for s in range(NSTREAM)]
            fresh = [o != stamp for o in olds]
            for s in range(NSTREAM):
                plsc.addupdate_scatter(hist, [cs[s]], ones, mask=fresh[s])
                plsc.store_scatter(markers[s], [midx[s]], stamp)
        return carry

    lax.fori_loop(0, quarter, row_body, 0)
    pltpu.sync_copy(hist, out_hbm.at[wid])


def _sc_hist(cat_flat, rows_per_lane, ann):
    mesh = plsc.VectorSubcoreMesh(
        core_axis_name="c", subcore_axis_name="s",
        num_cores=NC, num_subcores=NS)
    per_tile = rows_per_lane * L * ann
    run = pl.kernel(
        functools.partial(_sc_hist_body, rows_per_lane=rows_per_lane,
                          ann=ann),
        out_type=jax.ShapeDtypeStruct((NW, C_PAD), jnp.int32),
        mesh=mesh,
        scratch_types=(
            [pltpu.VMEM((per_tile,), jnp.int32)]
            + [pltpu.VMEM((L * C_PAD,), jnp.int32)
               for _ in range(NSTREAM)]
            + [pltpu.VMEM((C_PAD,), jnp.int32), pltpu.SemaphoreType.DMA]
        ),
        compiler_params=pltpu.CompilerParams(needs_layout_passes=False),
    )
    return run(cat_flat)


def _tc_idf_body(n_rows, counts_ref, out_ref):
    df = jnp.sum(counts_ref[...], axis=0, keepdims=True)
    df = df.astype(jnp.float32) + 1.0
    out_ref[...] = jnp.log((n_rows + 1) / df) + 1.0


def _tc_idf(counts, n_rows):
    return pl.pallas_call(
        functools.partial(_tc_idf_body, n_rows),
        out_shape=jax.ShapeDtypeStruct((1, C_PAD), jnp.float32),
    )(counts)


@jax.jit
def kernel(category_id):
    n_rows, ann = category_id.shape
    rows_per_lane = n_rows // (NW * L)
    counts = _sc_hist(category_id.reshape(-1), rows_per_lane, ann)
    weights = _tc_idf(counts, n_rows)
    return weights[0, :NUM_CLASSES]


# speedup vs baseline: 1.0528x; 1.0528x over previous
import functools

import jax
import jax.numpy as jnp
from jax import lax
from jax.experimental import pallas as pl
from jax.experimental.pallas import tpu as pltpu
from jax.experimental.pallas import tpu_sc as plsc

NUM_CLASSES = 1203
C_PAD = 1280
NC, NS, L = 2, 16, 16
NW = NC * NS
NSTREAM = 4
M_WORDS = 19456


def _sc_hist_body(cat_hbm, out_hbm, data, m0, m1, m2, m3, hist, sem,
                  *, rows_per_lane, ann):
    markers = (m0, m1, m2, m3)
    wid = lax.axis_index("s") * NC + lax.axis_index("c")
    per_tile = rows_per_lane * ann * L
    copy = pltpu.async_copy(
        cat_hbm.at[pl.ds(wid * per_tile, per_tile)], data, sem)

    iota = lax.iota(jnp.int32, 16)
    ones = jnp.ones((16,), jnp.int32)
    neg1 = jnp.full((16,), -1, jnp.int32)
    zero = jnp.zeros((16,), jnp.int32)

    def init_body(i, carry):
        for u in range(8):
            for m in markers:
                m[pl.ds(i * 128 + u * 16, 16)] = neg1
        return carry

    lax.fori_loop(0, M_WORDS // 128, init_body, 0)

    def hinit_body(i, carry):
        for u in range(8):
            hist[pl.ds(i * 128 + u * 16, 16)] = zero
        return carry

    lax.fori_loop(0, C_PAD // 128, hinit_body, 0)
    copy.wait()

    quarter = rows_per_lane // NSTREAM
    per_lane = rows_per_lane * ann
    lane_data = iota * per_lane

    def row_body(r, carry):
        rowv = (iota + r) & (quarter - 1)
        stamp = rowv
        dbase = [lane_data + (s * quarter) * ann + rowv * ann
                 for s in range(NSTREAM)]
        for j in range(ann):
            cs = [plsc.load_gather(data, [dbase[s] + j])
                  for s in range(NSTREAM)]
            midx = [(c << 4) + iota for c in cs]
            olds = [plsc.load_gather(markers[s], [midx[s]])
                    for s in range(NSTREAM)]
            fresh = [o != stamp for o in olds]
            for s in range(NSTREAM):
                plsc.addupdate_scatter(hist, [cs[s]], ones, mask=fresh[s])
                plsc.store_scatter(markers[s], [midx[s]], stamp)
        return carry

    lax.fori_loop(0, quarter, row_body, 0)
    pltpu.sync_copy(hist, out_hbm.at[wid])


def _sc_hist(cat_flat, rows_per_lane, ann):
    mesh = plsc.VectorSubcoreMesh(
        core_axis_name="c", subcore_axis_name="s",
        num_cores=NC, num_subcores=NS)
    per_tile = rows_per_lane * L * ann
    run = pl.kernel(
        functools.partial(_sc_hist_body, rows_per_lane=rows_per_lane,
                          ann=ann),
        out_type=jax.ShapeDtypeStruct((NW, C_PAD), jnp.int32),
        mesh=mesh,
        scratch_types=(
            [pltpu.VMEM((per_tile,), jnp.int32)]
            + [pltpu.VMEM((M_WORDS,), jnp.int32)
               for _ in range(NSTREAM)]
            + [pltpu.VMEM((C_PAD,), jnp.int32), pltpu.SemaphoreType.DMA]
        ),
        compiler_params=pltpu.CompilerParams(needs_layout_passes=False),
    )
    return run(cat_flat)


def _tc_idf_body(n_rows, counts_ref, out_ref):
    df = jnp.sum(counts_ref[...], axis=0, keepdims=True)
    df = df.astype(jnp.float32) + 1.0
    out_ref[...] = jnp.log((n_rows + 1) / df) + 1.0


def _tc_idf(counts, n_rows):
    return pl.pallas_call(
        functools.partial(_tc_idf_body, n_rows),
        out_shape=jax.ShapeDtypeStruct((1, C_PAD), jnp.float32),
    )(counts)


@jax.jit
def kernel(category_id):
    n_rows, ann = category_id.shape
    rows_per_lane = n_rows // (NW * L)
    counts = _sc_hist(category_id.reshape(-1), rows_per_lane, ann)
    weights = _tc_idf(counts, n_rows)
    return weights[0, :NUM_CLASSES]
